# baseline (speedup 1.0000x reference)
import functools
import jax
import jax.numpy as jnp
from jax import lax
from jax.experimental import pallas as pl
from jax.experimental.pallas import tpu as pltpu
from jax.experimental.pallas import tpu_sc as plsc

B, L, D1, D2, DM, MAXLEN = 4, 2048, 512, 1024, 2048, 4096
BLK = 128
NTOK = B * L
NPAD = 8448
NBLKS = NPAD // BLK
LBLK = L // BLK
NW = 32
EPS = 1e-5


def _gelu(v):
    return 0.5 * v * (1.0 + jax.lax.erf(v * (2.0 ** -0.5)))


def _sc_gather_rows(table, idx, ch):
    n = idx.shape[0]
    d = table.shape[1]
    rpw = n // NW
    nch = rpw // ch
    mesh = plsc.VectorSubcoreMesh(core_axis_name="c", subcore_axis_name="s")

    def body(table_hbm, idx_hbm, out_hbm, idx_v, b0, b1, s0, s1):
        wid = lax.axis_index("s") * 2 + lax.axis_index("c")
        base = wid * rpw
        pltpu.sync_copy(idx_hbm.at[pl.ds(base, rpw)], idx_v)
        bufs = (b0, b1)
        sems = (s0, s1)

        def start(c):
            return pltpu.async_copy(
                table_hbm.at[idx_v.at[pl.ds(c * ch, ch)]],
                bufs[c % 2], sems[c % 2])

        cp = start(0)
        for c in range(nch):
            nxt = start(c + 1) if c + 1 < nch else None
            cp.wait()
            pltpu.sync_copy(bufs[c % 2], out_hbm.at[pl.ds(base + c * ch, ch)])
            cp = nxt

    f = pl.kernel(
        body, mesh=mesh,
        out_type=jax.ShapeDtypeStruct((n, d), jnp.float32),
        scratch_types=[
            pltpu.VMEM((rpw,), jnp.int32),
            pltpu.VMEM((ch, d), jnp.float32),
            pltpu.VMEM((ch, d), jnp.float32),
            pltpu.SemaphoreType.DMA,
            pltpu.SemaphoreType.DMA,
        ])
    return f(table, idx)


def _mm_body(eids, x_ref, w1a, b1a, w1b, b1b, w2a, b2a, w2b, b2b, ttab, out_ref):
    i = pl.program_id(0)
    e = eids[i]

    @pl.when(e == 0)
    def _():
        h = _gelu(jnp.dot(x_ref[:, :D1], w1a[...],
                          preferred_element_type=jnp.float32) + b1a[...])
        out_ref[...] = (jnp.dot(h, w1b[...], preferred_element_type=jnp.float32)
                        + b1b[...] + ttab[0:1, :])

    @pl.when(e != 0)
    def _():
        h = _gelu(jnp.dot(x_ref[...], w2a[...],
                          preferred_element_type=jnp.float32) + b2a[...])
        out_ref[...] = (jnp.dot(h, w2b[...], preferred_element_type=jnp.float32)
                        + b2b[...] + ttab[1:2, :])


def _ln_body(y_ref, pos_ref, gamma, beta, out_ref):
    o = y_ref[...] + pos_ref[...]
    mu = jnp.mean(o, axis=-1, keepdims=True)
    c = o - mu
    var = jnp.mean(c * c, axis=-1, keepdims=True)
    out_ref[...] = c * jax.lax.rsqrt(var + EPS) * gamma[...] + beta[...]


def kernel(x, token_type_ids, W1a, b1a, W1b, b1b, W2a, b2a, W2b, b2b,
           type_table, pos_table, gamma, beta):
    xf = x.reshape(NTOK, D2)
    t = token_type_ids.reshape(NTOK)
    is0 = (t == 0).astype(jnp.int32)
    c0 = jnp.cumsum(is0)
    n0 = c0[-1]
    rank0 = c0 - is0
    is1 = 1 - is0
    c1 = jnp.cumsum(is1)
    rank1 = c1 - is1
    base1 = ((n0 + BLK - 1) // BLK) * BLK
    dest = jnp.where(is0 == 1, rank0, base1 + rank1).astype(jnp.int32)
    src = jnp.zeros((NPAD,), jnp.int32).at[dest].set(
        jnp.arange(NTOK, dtype=jnp.int32))
    eids = (jnp.arange(NBLKS, dtype=jnp.int32) * BLK >= base1).astype(jnp.int32)

    x_sorted = _sc_gather_rows(xf, src, ch=24)

    full = lambda s: pl.BlockSpec(s, lambda i, e: (0,) * len(s))
    y_sorted = pl.pallas_call(
        _mm_body,
        grid_spec=pltpu.PrefetchScalarGridSpec(
            num_scalar_prefetch=1,
            grid=(NBLKS,),
            in_specs=[
                pl.BlockSpec((BLK, D2), lambda i, e: (i, 0)),
                full((D1, DM)), full((1, DM)),
                full((DM, DM)), full((1, DM)),
                full((D2, DM)), full((1, DM)),
                full((DM, DM)), full((1, DM)),
                full((2, DM)),
            ],
            out_specs=pl.BlockSpec((BLK, DM), lambda i, e: (i, 0)),
        ),
        out_shape=jax.ShapeDtypeStruct((NPAD, DM), jnp.float32),
        compiler_params=pltpu.CompilerParams(
            dimension_semantics=("arbitrary",)),
    )(eids, x_sorted, W1a, b1a.reshape(1, DM), W1b, b1b.reshape(1, DM),
      W2a, b2a.reshape(1, DM), W2b, b2b.reshape(1, DM), type_table)

    y = _sc_gather_rows(y_sorted, dest, ch=16)

    out = pl.pallas_call(
        _ln_body,
        grid=(NTOK // BLK,),
        in_specs=[
            pl.BlockSpec((BLK, DM), lambda i: (i, 0)),
            pl.BlockSpec((BLK, DM), lambda i: (i % LBLK, 0)),
            pl.BlockSpec((1, DM), lambda i: (0, 0)),
            pl.BlockSpec((1, DM), lambda i: (0, 0)),
        ],
        out_specs=pl.BlockSpec((BLK, DM), lambda i: (i, 0)),
        out_shape=jax.ShapeDtypeStruct((NTOK, DM), jnp.float32),
        compiler_params=pltpu.CompilerParams(
            dimension_semantics=("arbitrary",)),
    )(y, pos_table, gamma.reshape(1, DM), beta.reshape(1, DM))
    return out.reshape(B, L, DM)
